# baseline (speedup 1.0000x reference)
import functools

import jax
import jax.numpy as jnp
from jax import lax
from jax.experimental import pallas as pl
from jax.experimental.pallas import tpu as pltpu
from jax.experimental.pallas import tpu_sc as plsc

POOL = 7
LANES = 16
NW = 32
MAXBAND = 5
MAXBLK = 5
NCG = 4
NGH = 8


def _roi_pool_sc(fm5, ttab, c_total):
    ntask = ttab.shape[0]
    tpw = ntask // NW
    nunit = NCG * tpw

    mesh = plsc.VectorSubcoreMesh(core_axis_name="c", subcore_axis_name="s")

    @functools.partial(
        pl.kernel,
        out_type=jax.ShapeDtypeStruct((ntask, POOL, c_total), jnp.float32),
        mesh=mesh,
        compiler_params=pltpu.CompilerParams(use_tc_tiling_on_sc=False),
        scratch_types=[
            pltpu.VMEM((2, MAXBAND, MAXBLK, 1, 8, 128), jnp.float32),
            pltpu.VMEM((2, POOL, c_total), jnp.float32),
            pltpu.VMEM((tpw, 2 * LANES), jnp.int32),
            pltpu.SemaphoreType.DMA((2,)),
            pltpu.SemaphoreType.DMA,
        ],
    )
    def k(fm5_hbm, ttab_hbm, out_hbm, band_v, outr_v, tt_v, sem, osem):
        wid = lax.axis_index("s") * 2 + lax.axis_index("c")
        pltpu.sync_copy(ttab_hbm.at[pl.ds(wid * tpw, tpw)], tt_v)

        def fire(u):
            kb = u // NCG
            cg = u % NCG
            par = u % 2
            vec = tt_v[kb, pl.ds(0, LANES)]
            rowstart = vec[0]
            rn = vec[1]
            nblk = vec[2]
            xblk = vec[4]
            for nb in range(1, MAXBLK + 1):
                @pl.when(nblk == nb)
                def _():
                    def jb(j, c):
                        pltpu.async_copy(
                            fm5_hbm.at[rowstart + j, pl.ds(xblk, nb),
                                       pl.ds(cg, 1)],
                            band_v.at[par, j, pl.ds(0, nb)],
                            sem.at[par])
                        return c
                    lax.fori_loop(0, rn, jb, 0)

        def drain(u):
            kb = u // NCG
            cg = u % NCG
            par = u % 2
            vec = tt_v[kb, pl.ds(0, LANES)]
            rowstart = vec[0]
            rn = vec[1]
            nblk = vec[2]
            xblk = vec[4]
            for nb in range(1, MAXBLK + 1):
                @pl.when(nblk == nb)
                def _():
                    def jb(j, c):
                        pltpu.make_async_copy(
                            fm5_hbm.at[rowstart + j, pl.ds(xblk, nb),
                                       pl.ds(cg, 1)],
                            band_v.at[par, j, pl.ds(0, nb)],
                            sem.at[par]).wait()
                        return c
                    lax.fori_loop(0, rn, jb, 0)

        fire(0)

        def body(u, carry):
            kb = u // NCG
            cg = u % NCG
            par = u % 2
            kpar = kb % 2

            @pl.when(u + 1 < nunit)
            def _():
                fire(u + 1)

            drain(u)

            vec0 = tt_v[kb, pl.ds(0, LANES)]
            vec1 = tt_v[kb, pl.ds(LANES, LANES)]
            rn = vec0[1]
            outpos = vec0[3]

            @pl.when((cg == 0) & (kb >= 2))
            def _():
                prev = tt_v[jnp.maximum(kb - 2, 0), pl.ds(0, LANES)]
                pltpu.make_async_copy(
                    outr_v.at[kpar], out_hbm.at[prev[3]], osem).wait()

            for px in range(POOL):
                c0 = vec1[px]
                cn = vec1[POOL + px]

                def row_body(j, accs):
                    def col_body(c, accs2):
                        blk = c // 8
                        sub = c % 8
                        return tuple(
                            jnp.maximum(
                                accs2[g],
                                band_v[par, j, blk, 0, sub,
                                       pl.ds(g * LANES, LANES)])
                            for g in range(NGH))
                    return lax.fori_loop(c0, c0 + cn, col_body, accs)

                neg = jnp.full((LANES,), -jnp.inf, jnp.float32)
                accs = lax.fori_loop(0, rn, row_body, (neg,) * NGH)
                for g in range(NGH):
                    outr_v[kpar, px,
                           pl.ds(cg * 128 + g * LANES, LANES)] = accs[g]

            @pl.when(cg == NCG - 1)
            def _():
                pltpu.async_copy(outr_v.at[kpar], out_hbm.at[outpos], osem)

            return carry

        lax.fori_loop(0, nunit, body, 0)

        def tail(t, carry):
            kb = tpw - 2 + t
            vec = tt_v[kb, pl.ds(0, LANES)]
            pltpu.make_async_copy(
                outr_v.at[kb % 2], out_hbm.at[vec[3]], osem).wait()
            return carry

        lax.fori_loop(0, 2, tail, 0)

    return k(fm5, ttab)


def kernel(x_maps, x_rois):
    B, H, W, C = x_maps.shape
    R = x_rois.shape[1]
    nroi = B * R
    y = x_rois[..., 0].astype(jnp.int32).reshape(-1)
    x = x_rois[..., 1].astype(jnp.int32).reshape(-1)
    h = x_rois[..., 2].astype(jnp.int32).reshape(-1)
    w = x_rois[..., 3].astype(jnp.int32).reshape(-1)
    b = jnp.arange(nroi, dtype=jnp.int32) // R

    p = jnp.arange(POOL, dtype=jnp.int32)
    y0 = (p * h[:, None]) // POOL
    y1 = ((p + 1) * h[:, None]) // POOL
    ys = jnp.maximum(y1 - y0, 1)
    x0 = (p * w[:, None]) // POOL
    x1 = ((p + 1) * w[:, None]) // POOL
    xs = jnp.maximum(x1 - x0, 1)

    ntask = nroi * POOL
    rowstart = ((b * H + y)[:, None] + y0)
    rn = ys
    xblk = x // 8
    xrem = x - xblk * 8
    nblk = (xrem + w + 7) // 8
    outpos = (jnp.arange(nroi, dtype=jnp.int32)[:, None] * POOL + p)
    zero = jnp.zeros((nroi, POOL), jnp.int32)

    def bcast(a):
        return jnp.broadcast_to(a[:, None], (nroi, POOL))

    vec0 = jnp.stack(
        [rowstart, rn, bcast(nblk), outpos, bcast(xblk)]
        + [zero] * 11, axis=-1)
    cs_rel = x0 + xrem[:, None]
    vec1 = jnp.concatenate(
        [jnp.broadcast_to(cs_rel[:, None, :], (nroi, POOL, POOL)),
         jnp.broadcast_to(xs[:, None, :], (nroi, POOL, POOL)),
         jnp.zeros((nroi, POOL, 2), jnp.int32)], axis=-1)
    ttab = jnp.concatenate([vec0, vec1], axis=-1).reshape(ntask, 2 * LANES)

    cost = (rn * bcast(nblk)).reshape(ntask)
    ranks = jnp.argsort(-cost)
    perm = ranks.reshape(ntask // NW, NW).T.reshape(ntask)
    ttab = ttab[perm].astype(jnp.int32)

    fm5 = x_maps.reshape(B * H, W // 8, 8, C // 128, 128)
    fm5 = fm5.transpose(0, 1, 3, 2, 4)

    out = _roi_pool_sc(fm5, ttab, C)
    return out.reshape(B, R, POOL, POOL, C)


# speedup vs baseline: 1.2487x; 1.2487x over previous
import functools

import jax
import jax.numpy as jnp
from jax import lax
from jax.experimental import pallas as pl
from jax.experimental.pallas import tpu as pltpu
from jax.experimental.pallas import tpu_sc as plsc

POOL = 7
LANES = 16
NW = 32
MAXBAND = 5
MAXBLK = 5
NGH = 16


def _roi_pool_sc(fm5, ttab, c_total):
    ntask = ttab.shape[0]
    tpw = ntask // NW
    nunit = 2 * tpw

    mesh = plsc.VectorSubcoreMesh(core_axis_name="c", subcore_axis_name="s")

    @functools.partial(
        pl.kernel,
        out_type=jax.ShapeDtypeStruct((ntask, POOL, c_total), jnp.float32),
        mesh=mesh,
        compiler_params=pltpu.CompilerParams(use_tc_tiling_on_sc=False),
        scratch_types=[
            pltpu.VMEM((2, MAXBAND, MAXBLK, 2, 8, 128), jnp.float32),
            pltpu.VMEM((2, POOL, c_total), jnp.float32),
            pltpu.VMEM((tpw, 2 * LANES), jnp.int32),
            pltpu.SemaphoreType.DMA((2,)),
            pltpu.SemaphoreType.DMA,
        ],
    )
    def k(fm5_hbm, ttab_hbm, out_hbm, band_v, outr_v, tt_v, sem, osem):
        wid = lax.axis_index("s") * 2 + lax.axis_index("c")
        pltpu.sync_copy(ttab_hbm.at[pl.ds(wid * tpw, tpw)], tt_v)

        def unit_dma(u, do_wait):
            kb = u // 2
            half = u % 2
            par = u % 2
            vec = tt_v[kb, pl.ds(0, LANES)]
            rowstart = vec[0]
            rn = vec[1]
            nblk = vec[2]
            xblk = vec[4]
            for rc in range(1, MAXBAND + 1):
                @pl.when(rn == rc)
                def _():
                    for nb in range(1, MAXBLK + 1):
                        @pl.when(nblk == nb)
                        def _():
                            c = pltpu.make_async_copy(
                                fm5_hbm.at[pl.ds(rowstart, rc),
                                           pl.ds(xblk, nb),
                                           pl.ds(half * 2, 2)],
                                band_v.at[par, pl.ds(0, rc), pl.ds(0, nb)],
                                sem.at[par])
                            if do_wait:
                                c.wait()
                            else:
                                c.start()

        unit_dma(0, False)

        def body(u, carry):
            kb = u // 2
            half = u % 2
            par = u % 2
            kpar = kb % 2

            @pl.when(u + 1 < nunit)
            def _():
                unit_dma(u + 1, False)

            unit_dma(u, True)

            vec0 = tt_v[kb, pl.ds(0, LANES)]
            vec1 = tt_v[kb, pl.ds(LANES, LANES)]
            rn = vec0[1]
            outpos = vec0[3]

            @pl.when((half == 0) & (kb >= 2))
            def _():
                prev = tt_v[jnp.maximum(kb - 2, 0), pl.ds(0, LANES)]
                pltpu.make_async_copy(
                    outr_v.at[kpar], out_hbm.at[prev[3]], osem).wait()

            for px in range(POOL):
                c0 = vec1[px]
                cn = vec1[POOL + px]

                def row_body(j, accs):
                    def col_body(c, accs2):
                        blk = c // 8
                        sub = c % 8
                        return tuple(
                            jnp.maximum(
                                accs2[g],
                                band_v[par, j, blk, g // 8, sub,
                                       pl.ds((g % 8) * LANES, LANES)])
                            for g in range(NGH))
                    return lax.fori_loop(c0, c0 + cn, col_body, accs)

                neg = jnp.full((LANES,), -jnp.inf, jnp.float32)
                accs = lax.fori_loop(0, rn, row_body, (neg,) * NGH)
                for g in range(NGH):
                    outr_v[kpar, px,
                           pl.ds(half * 256 + g * LANES, LANES)] = accs[g]

            @pl.when(half == 1)
            def _():
                pltpu.async_copy(outr_v.at[kpar], out_hbm.at[outpos], osem)

            return carry

        lax.fori_loop(0, nunit, body, 0)

        def tail(t, carry):
            kb = tpw - 2 + t
            vec = tt_v[kb, pl.ds(0, LANES)]
            pltpu.make_async_copy(
                outr_v.at[kb % 2], out_hbm.at[vec[3]], osem).wait()
            return carry

        lax.fori_loop(0, 2, tail, 0)

    return k(fm5, ttab)


def kernel(x_maps, x_rois):
    B, H, W, C = x_maps.shape
    R = x_rois.shape[1]
    nroi = B * R
    y = x_rois[..., 0].astype(jnp.int32).reshape(-1)
    x = x_rois[..., 1].astype(jnp.int32).reshape(-1)
    h = x_rois[..., 2].astype(jnp.int32).reshape(-1)
    w = x_rois[..., 3].astype(jnp.int32).reshape(-1)
    b = jnp.arange(nroi, dtype=jnp.int32) // R

    p = jnp.arange(POOL, dtype=jnp.int32)
    y0 = (p * h[:, None]) // POOL
    y1 = ((p + 1) * h[:, None]) // POOL
    ys = jnp.maximum(y1 - y0, 1)
    x0 = (p * w[:, None]) // POOL
    x1 = ((p + 1) * w[:, None]) // POOL
    xs = jnp.maximum(x1 - x0, 1)

    ntask = nroi * POOL
    rowstart = ((b * H + y)[:, None] + y0)
    rn = ys
    xblk = x // 8
    xrem = x - xblk * 8
    nblk = (xrem + w + 7) // 8
    outpos = (jnp.arange(nroi, dtype=jnp.int32)[:, None] * POOL + p)
    zero = jnp.zeros((nroi, POOL), jnp.int32)

    def bcast(a):
        return jnp.broadcast_to(a[:, None], (nroi, POOL))

    vec0 = jnp.stack(
        [rowstart, rn, bcast(nblk), outpos, bcast(xblk)]
        + [zero] * 11, axis=-1)
    cs_rel = x0 + xrem[:, None]
    vec1 = jnp.concatenate(
        [jnp.broadcast_to(cs_rel[:, None, :], (nroi, POOL, POOL)),
         jnp.broadcast_to(xs[:, None, :], (nroi, POOL, POOL)),
         jnp.zeros((nroi, POOL, 2), jnp.int32)], axis=-1)
    ttab = jnp.concatenate([vec0, vec1], axis=-1).reshape(ntask, 2 * LANES)

    cost = (rn * bcast(nblk)).reshape(ntask)
    ranks = jnp.argsort(-cost)
    perm = ranks.reshape(ntask // NW, NW).T.reshape(ntask)
    ttab = ttab[perm].astype(jnp.int32)

    fm5 = x_maps.reshape(B * H, W // 8, 8, C // 128, 128)
    fm5 = fm5.transpose(0, 1, 3, 2, 4)

    out = _roi_pool_sc(fm5, ttab, C)
    return out.reshape(B, R, POOL, POOL, C)


# speedup vs baseline: 1.3525x; 1.0832x over previous
import functools

import jax
import jax.numpy as jnp
from jax import lax
from jax.experimental import pallas as pl
from jax.experimental.pallas import tpu as pltpu
from jax.experimental.pallas import tpu_sc as plsc

POOL = 7
LANES = 16
NW = 32
MAXBAND = 5
MAXBLK = 5
NGH = 16
TPW = 32


def _roi_pool_sc(fm3, ttab, ntask, c_total):
    nunit = 2 * TPW

    mesh = plsc.VectorSubcoreMesh(core_axis_name="c", subcore_axis_name="s")

    @functools.partial(
        pl.kernel,
        out_type=jax.ShapeDtypeStruct((ntask, POOL, c_total), jnp.float32),
        mesh=mesh,
        compiler_params=pltpu.CompilerParams(use_tc_tiling_on_sc=True),
        scratch_types=[
            pltpu.VMEM((2, MAXBAND, MAXBLK * 8, 256), jnp.float32),
            pltpu.VMEM((2, POOL, c_total), jnp.float32),
            pltpu.VMEM((TPW, 2 * LANES), jnp.int32),
            pltpu.SemaphoreType.DMA((2,)),
            pltpu.SemaphoreType.DMA,
        ],
    )
    def k(fm3_hbm, ttab_hbm, out_hbm, band_v, outr_v, tt_v, sem, osem):
        wid = lax.axis_index("s") * 2 + lax.axis_index("c")
        pltpu.sync_copy(ttab_hbm.at[pl.ds(wid * TPW, TPW)], tt_v)

        def unit_dma(u, do_wait):
            kb = u // 2
            half = u % 2
            par = u % 2
            vec = tt_v[kb, pl.ds(0, LANES)]
            rowstart = vec[0]
            rn = vec[1]
            nblk = vec[2]
            xblk = vec[4]
            for rc in range(1, MAXBAND + 1):
                @pl.when(rn == rc)
                def _():
                    for nb in range(1, MAXBLK + 1):
                        @pl.when(nblk == nb)
                        def _():
                            c = pltpu.make_async_copy(
                                fm3_hbm.at[pl.ds(rowstart, rc),
                                           pl.ds(xblk * 8, nb * 8),
                                           pl.ds(half * 256, 256)],
                                band_v.at[par, pl.ds(0, rc), pl.ds(0, nb * 8)],
                                sem.at[par])
                            if do_wait:
                                c.wait()
                            else:
                                c.start()

        unit_dma(0, False)

        def body(u, carry):
            kb = u // 2
            half = u % 2
            par = u % 2
            kpar = kb % 2

            @pl.when(u + 1 < nunit)
            def _():
                unit_dma(u + 1, False)

            unit_dma(u, True)

            vec0 = tt_v[kb, pl.ds(0, LANES)]
            vec1 = tt_v[kb, pl.ds(LANES, LANES)]
            rn = vec0[1]
            outpos = vec0[3]

            @pl.when((half == 0) & (kb >= 2))
            def _():
                prev = tt_v[jnp.maximum(kb - 2, 0), pl.ds(0, LANES)]
                pltpu.make_async_copy(
                    outr_v.at[kpar], out_hbm.at[prev[3]], osem).wait()

            for px in range(POOL):
                c0 = vec1[px]
                cn = vec1[POOL + px]

                def row_body(j, accs):
                    def col_body(c, accs2):
                        return tuple(
                            jnp.maximum(
                                accs2[g],
                                band_v[par, j, c, pl.ds(g * LANES, LANES)])
                            for g in range(NGH))
                    return lax.fori_loop(c0, c0 + cn, col_body, accs)

                neg = jnp.full((LANES,), -jnp.inf, jnp.float32)
                accs = lax.fori_loop(0, rn, row_body, (neg,) * NGH)
                for g in range(NGH):
                    outr_v[kpar, px,
                           pl.ds(half * 256 + g * LANES, LANES)] = accs[g]

            @pl.when(half == 1)
            def _():
                pltpu.async_copy(outr_v.at[kpar], out_hbm.at[outpos], osem)

            return carry

        lax.fori_loop(0, nunit, body, 0)

        def tail(t, carry):
            kb = TPW - 2 + t
            vec = tt_v[kb, pl.ds(0, LANES)]
            pltpu.make_async_copy(
                outr_v.at[kb % 2], out_hbm.at[vec[3]], osem).wait()
            return carry

        lax.fori_loop(0, 2, tail, 0)

    return k(fm3, ttab)


def kernel(x_maps, x_rois):
    B, H, W, C = x_maps.shape
    R = x_rois.shape[1]
    nroi = B * R
    y = x_rois[..., 0].astype(jnp.int32).reshape(-1)
    x = x_rois[..., 1].astype(jnp.int32).reshape(-1)
    h = x_rois[..., 2].astype(jnp.int32).reshape(-1)
    w = x_rois[..., 3].astype(jnp.int32).reshape(-1)
    b = jnp.arange(nroi, dtype=jnp.int32) // R

    p = jnp.arange(POOL, dtype=jnp.int32)
    y0 = (p * h[:, None]) // POOL
    y1 = ((p + 1) * h[:, None]) // POOL
    ys = jnp.maximum(y1 - y0, 1)
    x0 = (p * w[:, None]) // POOL
    x1 = ((p + 1) * w[:, None]) // POOL
    xs = jnp.maximum(x1 - x0, 1)

    ntask = nroi * POOL
    rowstart = ((b * H + y)[:, None] + y0)
    rn = ys
    xblk = x // 8
    xrem = x - xblk * 8
    nblk = (xrem + w + 7) // 8
    outpos = (jnp.arange(nroi, dtype=jnp.int32)[:, None] * POOL + p)
    zero = jnp.zeros((nroi, POOL), jnp.int32)

    def bcast(a):
        return jnp.broadcast_to(a[:, None], (nroi, POOL))

    vec0 = jnp.stack(
        [rowstart, rn, bcast(nblk), outpos, bcast(xblk)]
        + [zero] * 11, axis=-1)
    cs_rel = x0 + xrem[:, None]
    vec1 = jnp.concatenate(
        [jnp.broadcast_to(cs_rel[:, None, :], (nroi, POOL, POOL)),
         jnp.broadcast_to(xs[:, None, :], (nroi, POOL, POOL)),
         jnp.zeros((nroi, POOL, 2), jnp.int32)], axis=-1)
    ttab = jnp.concatenate([vec0, vec1], axis=-1).reshape(ntask, 2 * LANES)

    cost = (rn * bcast(nblk)).reshape(ntask)
    ranks = jnp.argsort(-cost)
    perm = ranks.reshape(ntask // NW, NW).T.reshape(NW, ntask // NW)
    perm = jnp.concatenate([perm, perm[:, -4:]], axis=1).reshape(-1)
    ttab = ttab[perm].astype(jnp.int32)

    fm3 = x_maps.reshape(B * H, W, C)
    out = _roi_pool_sc(fm3, ttab, ntask, C)
    return out.reshape(B, R, POOL, POOL, C)
